# baseline (speedup 1.0000x reference)
import jax
import jax.numpy as jnp
from jax import lax
from jax.experimental import pallas as pl
from jax.experimental.pallas import tpu as pltpu
from jax.experimental.pallas import tpu_sc as plsc

N = 10000
E = 320000
D = 128

NC = 2
NS = 16
NW = NC * NS
K = 16
CHUNKS = E // (NW * K)
E_PER_TILE = CHUNKS * K
NB = 6
NQ = 12
PEEL = 13
ROWS_PER_TILE = 624
ROWS_TAIL = N - NS * ROWS_PER_TILE


def _sc_body(x_hbm, ei_hbm, attr_hbm, out_hbm,
             sidx, didx, rows, attr, acc,
             sem_idx, sem_in, sem_s):
    c = lax.axis_index("c")
    s = lax.axis_index("s")
    tid = c * NS + s
    tile_base = tid * E_PER_TILE

    def issue_idx(eb, q):
        pltpu.async_copy(ei_hbm.at[pl.ds(eb, K)], sidx[q], sem_idx[q])
        pltpu.async_copy(ei_hbm.at[pl.ds(E + eb, K)], didx[q], sem_idx[q])

    issue_idx(tile_base, 0)
    issue_idx(tile_base + K, 1)

    zv = jnp.zeros((16,), jnp.float32)

    @pl.loop(0, K)
    def _zrow(r):
        for l in range(D // 16):
            rows[0][r, pl.ds(l * 16, 16)] = zv

    r0 = s * ROWS_PER_TILE
    NZ = ROWS_PER_TILE // K
    for m in range(NZ):
        pltpu.async_copy(rows[0], acc.at[pl.ds(r0 + m * K, K)], sem_s[m % NB])

    @pl.when(s == NS - 1)
    def _zero_tail():
        pltpu.async_copy(rows[0], acc.at[pl.ds(NS * ROWS_PER_TILE, ROWS_TAIL)],
                         sem_s[NZ % NB])

    for m in range(NZ):
        pltpu.make_async_copy(rows[0], acc.at[pl.ds(r0, K)],
                              sem_s[m % NB]).wait()

    @pl.when(s == NS - 1)
    def _zero_tail_wait():
        pltpu.make_async_copy(rows[0], acc.at[pl.ds(r0, ROWS_TAIL)],
                              sem_s[NZ % NB]).wait()

    plsc.subcore_barrier()

    def idx_wait(q):
        pltpu.make_async_copy(ei_hbm.at[pl.ds(0, K)], sidx[q], sem_idx[q]).wait()
        pltpu.make_async_copy(ei_hbm.at[pl.ds(0, K)], didx[q], sem_idx[q]).wait()

    def issue_in(eb, b, q):
        pltpu.async_copy(x_hbm.at[sidx[q]], rows[b], sem_in[b])
        pltpu.async_copy(attr_hbm.at[pl.ds(eb, K)], attr[b], sem_in[b])

    def in_wait(b, q):
        pltpu.make_async_copy(x_hbm.at[sidx[q]], rows[b], sem_in[b]).wait()
        pltpu.make_async_copy(attr_hbm.at[pl.ds(0, K)], attr[b], sem_in[b]).wait()

    def issue_scatter(b, q):
        pltpu.async_copy(rows[b], acc.at[didx[q]], sem_s[b], add=True)
        pltpu.async_copy(attr[b], acc.at[didx[q]], sem_s[b], add=True)

    def scatter_wait(b):
        pltpu.make_async_copy(rows[b], acc.at[didx[0]], sem_s[b]).wait()
        pltpu.make_async_copy(rows[b], acc.at[didx[0]], sem_s[b]).wait()

    def body(j, eb, peb, drain_prev_scatter, wait_prev_in):
        q, b = j % NQ, j % NB
        if drain_prev_scatter:
            scatter_wait(b)
        issue_idx(pl.multiple_of(peb, 8), (j + 2) % NQ)
        idx_wait(q)
        issue_in(pl.multiple_of(eb, 8), b, q)
        if wait_prev_in:
            in_wait((j - 1) % NB, (j - 1) % NQ)
            issue_scatter((j - 1) % NB, (j - 1) % NQ)

    for j in range(PEEL):
        body(j, tile_base + j * K, tile_base + (j + 2) * K,
             drain_prev_scatter=(j >= NB), wait_prev_in=(j >= 1))

    @pl.loop(0, (CHUNKS - PEEL) // NQ)
    def _group(g):
        j0 = PEEL + NQ * g
        for u in range(NQ):
            j = j0 + u
            nj = j + 2
            peb = jnp.where(nj < CHUNKS, tile_base + nj * K, 0)
            body(PEEL + u, tile_base + j * K, peb, True, True)

    last = CHUNKS - 1
    in_wait(last % NB, last % NQ)
    issue_scatter(last % NB, last % NQ)
    idx_wait((last + 1) % NQ)
    idx_wait((last + 2) % NQ)
    for d in range(NB):
        scatter_wait((last - NB + 1 + d) % NB)

    plsc.subcore_barrier()
    pltpu.sync_copy(acc.at[pl.ds(r0, ROWS_PER_TILE)],
                    out_hbm.at[c, pl.ds(r0, ROWS_PER_TILE)])

    @pl.when(s == NS - 1)
    def _write_tail():
        pltpu.sync_copy(acc.at[pl.ds(NS * ROWS_PER_TILE, ROWS_TAIL)],
                        out_hbm.at[c, pl.ds(NS * ROWS_PER_TILE, ROWS_TAIL)])


_sc_scatter = pl.kernel(
    _sc_body,
    out_type=jax.ShapeDtypeStruct((NC, N, D), jnp.float32),
    mesh=plsc.VectorSubcoreMesh(core_axis_name="c", subcore_axis_name="s"),
    scratch_types=[
        [pltpu.VMEM((K,), jnp.int32) for _ in range(NQ)],
        [pltpu.VMEM((K,), jnp.int32) for _ in range(NQ)],
        [pltpu.VMEM((K, D), jnp.float32) for _ in range(NB)],
        [pltpu.VMEM((K, D), jnp.float32) for _ in range(NB)],
        pltpu.VMEM_SHARED((N, D), jnp.float32),
        [pltpu.SemaphoreType.DMA for _ in range(NQ)],
        [pltpu.SemaphoreType.DMA for _ in range(NB)],
        [pltpu.SemaphoreType.DMA for _ in range(NB)],
    ],
)


def _combine_body(x_ref, p_ref, o_ref):
    o_ref[...] = x_ref[...] + p_ref[0] + p_ref[1]


_combine = pl.pallas_call(
    _combine_body,
    grid=(10,),
    in_specs=[pl.BlockSpec((1000, D), lambda i: (i, 0)),
              pl.BlockSpec((NC, 1000, D), lambda i: (0, i, 0))],
    out_specs=pl.BlockSpec((1000, D), lambda i: (i, 0)),
    out_shape=jax.ShapeDtypeStruct((N, D), jnp.float32),
)


@jax.jit
def kernel(x, edge_attr, edge_index):
    partials = _sc_scatter(x, edge_index.reshape(2 * E), edge_attr)
    return _combine(x, partials)


# speedup vs baseline: 1.5787x; 1.5787x over previous
import jax
import jax.numpy as jnp
from jax import lax
from jax.experimental import pallas as pl
from jax.experimental.pallas import tpu as pltpu
from jax.experimental.pallas import tpu_sc as plsc

N = 10000
E = 320000
D = 128

NC = 2
NS = 16
NW = NC * NS
K = 40
CHUNKS = E // (NW * K)
E_PER_TILE = CHUNKS * K
NBR = 4
NBA = 3
NQ = 6
UNROLL = 12
ROWS_PER_TILE = 624
ROWS_TAIL = N - NS * ROWS_PER_TILE


def _sc_body(x_hbm, ei_hbm, attr_hbm, out_hbm,
             sidx, didx, rows, attr, acc,
             sem_idx, sem_gr, sem_ga, sem_sr, sem_sa):
    c = lax.axis_index("c")
    s = lax.axis_index("s")
    tid = c * NS + s
    tile_base = tid * E_PER_TILE

    def issue_idx(eb, q):
        pltpu.async_copy(ei_hbm.at[pl.ds(eb, K)], sidx[q], sem_idx[q])
        pltpu.async_copy(ei_hbm.at[pl.ds(E + eb, K)], didx[q], sem_idx[q])

    issue_idx(tile_base, 0)
    issue_idx(tile_base + K, 1)

    zv = jnp.zeros((16,), jnp.float32)

    @pl.loop(0, K)
    def _zrow(r):
        for l in range(D // 16):
            rows[0][r, pl.ds(l * 16, 16)] = zv

    r0 = s * ROWS_PER_TILE
    for m in range(ROWS_PER_TILE // K):
        pltpu.sync_copy(rows[0], acc.at[pl.ds(r0 + m * K, K)])
    rem = ROWS_PER_TILE % K
    pltpu.sync_copy(rows[0].at[pl.ds(0, rem)],
                    acc.at[pl.ds(r0 + ROWS_PER_TILE - rem, rem)])

    @pl.when(s == NS - 1)
    def _zero_tail():
        pltpu.sync_copy(rows[0].at[pl.ds(0, ROWS_TAIL)],
                        acc.at[pl.ds(NS * ROWS_PER_TILE, ROWS_TAIL)])

    plsc.subcore_barrier()

    def idx_wait(q):
        pltpu.make_async_copy(ei_hbm.at[pl.ds(0, K)], sidx[q], sem_idx[q]).wait()
        pltpu.make_async_copy(ei_hbm.at[pl.ds(0, K)], didx[q], sem_idx[q]).wait()

    def issue_in(eb, j):
        pltpu.async_copy(x_hbm.at[sidx[j % NQ]], rows[j % NBR], sem_gr[j % NBR])
        pltpu.async_copy(attr_hbm.at[pl.ds(eb, K)], attr[j % NBA],
                         sem_ga[j % NBA])

    def in_wait(j):
        pltpu.make_async_copy(x_hbm.at[sidx[j % NQ]], rows[j % NBR],
                              sem_gr[j % NBR]).wait()
        pltpu.make_async_copy(attr_hbm.at[pl.ds(0, K)], attr[j % NBA],
                              sem_ga[j % NBA]).wait()

    def issue_scatter(j):
        q = j % NQ
        pltpu.async_copy(rows[j % NBR], acc.at[didx[q]], sem_sr[j % NBR],
                         add=True)
        pltpu.async_copy(attr[j % NBA], acc.at[didx[q]], sem_sa[j % NBA],
                         add=True)

    def rows_scatter_wait(br):
        pltpu.make_async_copy(rows[br], acc.at[didx[0]], sem_sr[br]).wait()

    def attr_scatter_wait(ba):
        pltpu.make_async_copy(attr[ba], acc.at[didx[0]], sem_sa[ba]).wait()

    def body(j, eb, peb, wait_prev_in):
        if j >= NBR:
            rows_scatter_wait(j % NBR)
        if j >= NBA:
            attr_scatter_wait(j % NBA)
        issue_idx(pl.multiple_of(peb, 8), (j + 2) % NQ)
        idx_wait(j % NQ)
        issue_in(pl.multiple_of(eb, 8), j)
        if wait_prev_in:
            in_wait(j - 1)
            issue_scatter(j - 1)

    PEEL = 10
    for j in range(PEEL):
        body(j, tile_base + j * K, tile_base + (j + 2) * K,
             wait_prev_in=(j >= 1))

    @pl.loop(0, (CHUNKS - PEEL) // UNROLL)
    def _group(g):
        j0 = PEEL + UNROLL * g
        for u in range(UNROLL):
            j = j0 + u
            nj = j + 2
            peb = jnp.where(nj < CHUNKS, tile_base + nj * K, 0)
            body(PEEL + u, tile_base + j * K, peb, True)

    last = CHUNKS - 1
    in_wait(last)
    issue_scatter(last)
    idx_wait((last + 1) % NQ)
    idx_wait((last + 2) % NQ)
    for d in range(NBR):
        rows_scatter_wait((last - NBR + 1 + d) % NBR)
    for d in range(NBA):
        attr_scatter_wait((last - NBA + 1 + d) % NBA)

    plsc.subcore_barrier()
    pltpu.sync_copy(acc.at[pl.ds(r0, ROWS_PER_TILE)],
                    out_hbm.at[c, pl.ds(r0, ROWS_PER_TILE)])

    @pl.when(s == NS - 1)
    def _write_tail():
        pltpu.sync_copy(acc.at[pl.ds(NS * ROWS_PER_TILE, ROWS_TAIL)],
                        out_hbm.at[c, pl.ds(NS * ROWS_PER_TILE, ROWS_TAIL)])


_sc_scatter = pl.kernel(
    _sc_body,
    out_type=jax.ShapeDtypeStruct((NC, N, D), jnp.float32),
    mesh=plsc.VectorSubcoreMesh(core_axis_name="c", subcore_axis_name="s"),
    scratch_types=[
        [pltpu.VMEM((K,), jnp.int32) for _ in range(NQ)],
        [pltpu.VMEM((K,), jnp.int32) for _ in range(NQ)],
        [pltpu.VMEM((K, D), jnp.float32) for _ in range(NBR)],
        [pltpu.VMEM((K, D), jnp.float32) for _ in range(NBA)],
        pltpu.VMEM_SHARED((N, D), jnp.float32),
        [pltpu.SemaphoreType.DMA for _ in range(NQ)],
        [pltpu.SemaphoreType.DMA for _ in range(NBR)],
        [pltpu.SemaphoreType.DMA for _ in range(NBA)],
        [pltpu.SemaphoreType.DMA for _ in range(NBR)],
        [pltpu.SemaphoreType.DMA for _ in range(NBA)],
    ],
)


def _combine_body(x_ref, p_ref, o_ref):
    o_ref[...] = x_ref[...] + p_ref[0] + p_ref[1]


_combine = pl.pallas_call(
    _combine_body,
    grid=(10,),
    in_specs=[pl.BlockSpec((1000, D), lambda i: (i, 0)),
              pl.BlockSpec((NC, 1000, D), lambda i: (0, i, 0))],
    out_specs=pl.BlockSpec((1000, D), lambda i: (i, 0)),
    out_shape=jax.ShapeDtypeStruct((N, D), jnp.float32),
)


@jax.jit
def kernel(x, edge_attr, edge_index):
    partials = _sc_scatter(x, edge_index.reshape(2 * E), edge_attr)
    return _combine(x, partials)
